# initial kernel scaffold (unmeasured)
import jax
import jax.numpy as jnp
from jax import lax
from jax.experimental import pallas as pl
from jax.experimental.pallas import tpu as pltpu

T = 1024
V_SHARD = 8192
D = 1024
V_CHUNK = 2048


def kernel(ids, E):
    def body(ids_ref, e_ref, out_ref, recv_ref, send_sem, recv_sem):
        my_x = lax.axis_index("x")
        my_y = lax.axis_index("y")
        my_z = lax.axis_index("z")

        ids_row = ids_ref[...].reshape(1, T)
        local_ids = ids_row - my_x * V_SHARD

        for k in range(V_SHARD // V_CHUNK):
            row_iota = lax.broadcasted_iota(jnp.int32, (V_CHUNK, T), 0)
            onehot_t = (row_iota + (k * V_CHUNK) == local_ids).astype(jnp.bfloat16)
            e_chunk = e_ref[k * V_CHUNK : (k + 1) * V_CHUNK, :].astype(jnp.bfloat16)
            part = lax.dot_general(
                onehot_t,
                e_chunk,
                (((0,), (0,)), ((), ())),
                preferred_element_type=jnp.float32,
            )
            if k == 0:
                out_ref[...] = part
            else:
                out_ref[...] += part

        rdma = pltpu.make_async_remote_copy(
            src_ref=out_ref,
            dst_ref=recv_ref,
            send_sem=send_sem,
            recv_sem=recv_sem,
            device_id=(1 - my_x, my_y, my_z),
            device_id_type=pl.DeviceIdType.MESH,
        )
        rdma.start()
        rdma.wait()
        out_ref[...] += recv_ref[...]

    return pl.pallas_call(
        body,
        out_shape=jax.ShapeDtypeStruct((T, D), jnp.float32),
        in_specs=[
            pl.BlockSpec(memory_space=pltpu.VMEM),
            pl.BlockSpec(memory_space=pltpu.VMEM),
        ],
        out_specs=pl.BlockSpec(memory_space=pltpu.VMEM),
        scratch_shapes=[
            pltpu.VMEM((T, D), jnp.float32),
            pltpu.SemaphoreType.DMA,
            pltpu.SemaphoreType.DMA,
        ],
        compiler_params=pltpu.CompilerParams(collective_id=0),
    )(ids, E)


# baseline (device time: 89314 ns/iter reference)
import jax
import jax.numpy as jnp
from jax import lax
from jax.experimental import pallas as pl
from jax.experimental.pallas import tpu as pltpu

T = 1024
V_SHARD = 8192
D = 1024
V_CHUNK = 2048


def kernel(ids, E):
    def body(ids_ref, e_ref, out_ref, recv_ref, send_sem, recv_sem):
        my_x = lax.axis_index("x")
        my_y = lax.axis_index("y")
        my_z = lax.axis_index("z")

        ids_row = ids_ref[...].reshape(1, T)
        local_ids = ids_row - my_x * V_SHARD

        for k in range(V_SHARD // V_CHUNK):
            row_iota = lax.broadcasted_iota(jnp.int32, (V_CHUNK, T), 0)
            onehot_t = (row_iota + (k * V_CHUNK) == local_ids).astype(jnp.bfloat16)
            e_chunk = e_ref[k * V_CHUNK : (k + 1) * V_CHUNK, :].astype(jnp.bfloat16)
            part = lax.dot_general(
                onehot_t,
                e_chunk,
                (((0,), (0,)), ((), ())),
                preferred_element_type=jnp.float32,
            )
            if k == 0:
                out_ref[...] = part
            else:
                out_ref[...] += part

        rdma = pltpu.make_async_remote_copy(
            src_ref=out_ref,
            dst_ref=recv_ref,
            send_sem=send_sem,
            recv_sem=recv_sem,
            device_id=(1 - my_x, my_y, my_z),
            device_id_type=pl.DeviceIdType.MESH,
        )
        rdma.start()
        rdma.wait()
        out_ref[...] += recv_ref[...]

    return pl.pallas_call(
        body,
        out_shape=jax.ShapeDtypeStruct((T, D), jnp.float32),
        in_specs=[
            pl.BlockSpec(memory_space=pltpu.VMEM),
            pl.BlockSpec(memory_space=pltpu.VMEM),
        ],
        out_specs=pl.BlockSpec(memory_space=pltpu.VMEM),
        scratch_shapes=[
            pltpu.VMEM((T, D), jnp.float32),
            pltpu.SemaphoreType.DMA,
            pltpu.SemaphoreType.DMA,
        ],
        compiler_params=pltpu.CompilerParams(
            vmem_limit_bytes=100 * 1024 * 1024,
        ),
    )(ids, E)


# device time: 62995 ns/iter; 1.4178x vs baseline; 1.4178x over previous
import jax
import jax.numpy as jnp
from jax import lax
from jax.experimental import pallas as pl
from jax.experimental.pallas import tpu as pltpu

T = 1024
V_SHARD = 8192
D = 1024
TC = T // 4
DH = D // 2


def kernel(ids, E):
    def body(ids_ref, e_ref, out_ref, recv_ref, send_sems, recv_sems):
        my_x = lax.axis_index("x")
        my_y = lax.axis_index("y")
        my_z = lax.axis_index("z")

        x_peer = (1 - my_x, my_y, my_z)
        y_peer = (my_x, 1 - my_y, my_z)
        z_peer = (my_x, my_y, 1 - my_z)

        c = 2 * my_z + my_y
        c_y = 2 * my_z + (1 - my_y)
        c_z = 2 * (1 - my_z) + my_y
        r, r_y, r_z = c * TC, c_y * TC, c_z * TC

        chunk_ids = ids_ref[pl.ds(c * TC, TC)].reshape(1, TC)
        local_ids = chunk_ids - my_x * V_SHARD
        row_iota = lax.broadcasted_iota(jnp.int32, (V_SHARD, TC), 0)
        onehot_t = (row_iota == local_ids).astype(jnp.bfloat16)
        partial = lax.dot_general(
            onehot_t,
            e_ref[...].astype(jnp.bfloat16),
            (((0,), (0,)), ((), ())),
            preferred_element_type=jnp.float32,
        )
        out_ref[pl.ds(r, TC), :] = partial

        def rdma(src, dst, sem_idx, peer):
            return pltpu.make_async_remote_copy(
                src_ref=src,
                dst_ref=dst,
                send_sem=send_sems.at[sem_idx],
                recv_sem=recv_sems.at[sem_idx],
                device_id=peer,
                device_id_type=pl.DeviceIdType.MESH,
            )

        rx = rdma(out_ref.at[pl.ds(r, TC), :], recv_ref, 0, x_peer)
        rx.start()
        rx.wait()
        out_ref[pl.ds(r, TC), :] += recv_ref[...]

        g1y = rdma(
            out_ref.at[pl.ds(r, TC), pl.ds(0, DH)],
            out_ref.at[pl.ds(r, TC), pl.ds(0, DH)],
            1, y_peer,
        )
        g1z = rdma(
            out_ref.at[pl.ds(r, TC), pl.ds(DH, DH)],
            out_ref.at[pl.ds(r, TC), pl.ds(DH, DH)],
            2, z_peer,
        )
        g1y.start()
        g1z.start()
        g1y.wait()
        g1z.wait()

        g2y_a = rdma(
            out_ref.at[pl.ds(r, TC), pl.ds(DH, DH)],
            out_ref.at[pl.ds(r, TC), pl.ds(DH, DH)],
            3, y_peer,
        )
        g2y_b = rdma(
            out_ref.at[pl.ds(r_z, TC), pl.ds(DH, DH)],
            out_ref.at[pl.ds(r_z, TC), pl.ds(DH, DH)],
            4, y_peer,
        )
        g2z_a = rdma(
            out_ref.at[pl.ds(r, TC), pl.ds(0, DH)],
            out_ref.at[pl.ds(r, TC), pl.ds(0, DH)],
            5, z_peer,
        )
        g2z_b = rdma(
            out_ref.at[pl.ds(r_y, TC), pl.ds(0, DH)],
            out_ref.at[pl.ds(r_y, TC), pl.ds(0, DH)],
            6, z_peer,
        )
        g2y_a.start()
        g2y_b.start()
        g2z_a.start()
        g2z_b.start()
        g2y_a.wait()
        g2y_b.wait()
        g2z_a.wait()
        g2z_b.wait()

    return pl.pallas_call(
        body,
        out_shape=jax.ShapeDtypeStruct((T, D), jnp.float32),
        in_specs=[
            pl.BlockSpec(memory_space=pltpu.VMEM),
            pl.BlockSpec(memory_space=pltpu.VMEM),
        ],
        out_specs=pl.BlockSpec(memory_space=pltpu.VMEM),
        scratch_shapes=[
            pltpu.VMEM((TC, D), jnp.float32),
            pltpu.SemaphoreType.DMA((7,)),
            pltpu.SemaphoreType.DMA((7,)),
        ],
        compiler_params=pltpu.CompilerParams(
            vmem_limit_bytes=100 * 1024 * 1024,
        ),
    )(ids, E)
